# baseline (speedup 1.0000x reference)
import functools

import jax
import jax.numpy as jnp
from jax import lax
from jax.experimental import pallas as pl
from jax.experimental.pallas import tpu as pltpu
from jax.experimental.pallas import tpu_sc as plsc

N = 10000
D = 256
HD = 128
E = 160000
CH = 128
NT = 16
NC = 2
NCH = E // CH
CPT0 = 79
NROW_PAD = N
RPT = 624
RPT_LAST = N - RPT * (NT - 1)
NG = 16
LAYERS = 3


def _prep_edges(ei):
    return jnp.ravel(ei[1]), jnp.ravel(ei[0])


def _sc_chain(x, dst3, src3):
    mesh = plsc.VectorSubcoreMesh(core_axis_name="c", subcore_axis_name="s")

    @functools.partial(
        pl.kernel,
        out_type=(jax.ShapeDtypeStruct((N, D), jnp.float32),
                  jax.ShapeDtypeStruct((N, D), jnp.float32),
                  jax.ShapeDtypeStruct((N, D), jnp.float32)),
        mesh=mesh,
        scratch_types=[
            pltpu.VMEM((CH,), jnp.int32),
            pltpu.VMEM((CH,), jnp.int32),
            pltpu.VMEM((CH,), jnp.int32),
            pltpu.VMEM((CH,), jnp.int32),
            pltpu.VMEM((CH,), jnp.int32),
            pltpu.VMEM((CH,), jnp.int32),
            pltpu.VMEM_SHARED((NROW_PAD, HD), jnp.float32),
            pltpu.VMEM((CH, HD), jnp.float32),
            pltpu.VMEM((CH, HD), jnp.float32),
            pltpu.VMEM((CH, HD), jnp.float32),
            pltpu.SemaphoreType.DMA,
            pltpu.SemaphoreType.DMA,
            pltpu.SemaphoreType.DMA,
            pltpu.SemaphoreType.DMA,
            pltpu.SemaphoreType.DMA,
            pltpu.SemaphoreType.DMA,
            pltpu.SemaphoreType.DMA,
            pltpu.SemaphoreType.DMA,
            pltpu.SemaphoreType.DMA,
        ],
    )
    def chain(x_hbm, dst_hbm, src_hbm, out_hbm, t1_hbm, t2_hbm, dstb0, dstb1,
              dstb2, srcb0, srcb1, srcb2, acc, rows0, rows1, rows2, gsem0,
              gsem1, gsem2, ssem0, ssem1, ssem2, dsem0, dsem1, dsem2):
        c = lax.axis_index("c")
        s = lax.axis_index("s")
        col = c * HD
        nc = 78 + (s < 2).astype(jnp.int32)
        ebase = (s * 78 + jnp.minimum(s, 2)) * CH

        def fetch_src(j, buf, sem):
            pltpu.async_copy(src_hbm.at[pl.ds(ebase + j * CH, CH)], buf, sem)

        def fetch_src_wait(j, buf, sem):
            pltpu.make_async_copy(
                src_hbm.at[pl.ds(ebase + j * CH, CH)], buf, sem).wait()

        def fetch_dst(j, buf, sem):
            pltpu.async_copy(dst_hbm.at[pl.ds(ebase + j * CH, CH)], buf, sem)

        def fetch_dst_wait(j, buf, sem):
            pltpu.make_async_copy(
                dst_hbm.at[pl.ds(ebase + j * CH, CH)], buf, sem).wait()

        @pl.when(s < NT - 1)
        def _():
            pltpu.sync_copy(x_hbm.at[pl.ds(s * RPT, RPT), pl.ds(col, HD)],
                            acc.at[pl.ds(s * RPT, RPT)])

        @pl.when(s == NT - 1)
        def _():
            pltpu.sync_copy(
                x_hbm.at[pl.ds((NT - 1) * RPT, RPT_LAST), pl.ds(col, HD)],
                acc.at[pl.ds((NT - 1) * RPT, RPT_LAST)])

        bufs = ((dstb0, srcb0, rows0, gsem0, ssem0, dsem0),
                (dstb1, srcb1, rows1, gsem1, ssem1, dsem1),
                (dstb2, srcb2, rows2, gsem2, ssem2, dsem2))

        def run_layer(xin_hbm, xout_hbm):
            def gather(db, buf, sem):
                return pltpu.async_copy(
                    xin_hbm.at[db, pl.ds(col, HD)], buf, sem)

            def gather_wait(db, buf, sem):
                pltpu.make_async_copy(
                    xin_hbm.at[db, pl.ds(col, HD)], buf, sem).wait()

            for k, (db, sb, rb, gs, ss, ds) in enumerate(bufs):
                pltpu.sync_copy(dst_hbm.at[pl.ds(ebase + k * CH, CH)], db)
                fetch_src(k, sb, ss)
                gather(db, rb, gs)
            plsc.subcore_barrier()

            @pl.loop(0, nc, step=3)
            def _(j):
                for k, (db, sb, rb, gs, ss, ds) in enumerate(bufs):
                    @pl.when(j + k < nc)
                    def _(k=k, db=db, sb=sb, rb=rb, gs=gs, ss=ss, ds=ds):
                        gather_wait(db, rb, gs)
                        fetch_src_wait(j + k, sb, ss)
                        pltpu.sync_copy(rb, acc.at[sb], add=True)

                        @pl.when(j + k + 3 < nc)
                        def _():
                            fetch_src(j + k + 3, sb, ss)
                            fetch_dst(j + k + 3, db, ds)

                for k, (db, sb, rb, gs, ss, ds) in enumerate(bufs):
                    @pl.when(j + k + 3 < nc)
                    def _(k=k, db=db, sb=sb, rb=rb, gs=gs, ss=ss, ds=ds):
                        fetch_dst_wait(j + k + 3, db, ds)
                        gather(db, rb, gs)

            plsc.subcore_barrier()

            @pl.when(s < NT - 1)
            def _():
                pltpu.sync_copy(
                    acc.at[pl.ds(s * RPT, RPT)],
                    xout_hbm.at[pl.ds(s * RPT, RPT), pl.ds(col, HD)])

            @pl.when(s == NT - 1)
            def _():
                pltpu.sync_copy(
                    acc.at[pl.ds((NT - 1) * RPT, RPT_LAST)],
                    xout_hbm.at[pl.ds((NT - 1) * RPT, RPT_LAST),
                                pl.ds(col, HD)])

            plsc.subcore_barrier()

        run_layer(x_hbm, t1_hbm)
        run_layer(t1_hbm, t2_hbm)
        run_layer(t2_hbm, out_hbm)

    return chain(x, dst3, src3)[0]


BR = 400


def _matmul1(fw, w1t, b):
    def body(fw_ref, w1_ref, b_ref, out_ref):
        out_ref[...] = jnp.dot(fw_ref[...], w1_ref[...],
                               preferred_element_type=jnp.float32) \
            + b_ref[...][None, :]

    return pl.pallas_call(
        body,
        grid=(N // BR,),
        in_specs=[
            pl.BlockSpec((BR, D), lambda i: (i, 0)),
            pl.BlockSpec((D, D), lambda i: (0, 0)),
            pl.BlockSpec((D,), lambda i: (0,)),
        ],
        out_specs=pl.BlockSpec((BR, D), lambda i: (i, 0)),
        out_shape=jax.ShapeDtypeStruct((N, D), jnp.float32),
    )(fw, w1t, b)


def _merge(part, bw, w2t, batch):
    def body(pt_ref, bw_ref, w2_ref, bt_ref, hout_ref, gh_ref):
        i = pl.program_id(0)
        hb = jnp.dot(bw_ref[...], w2_ref[...],
                     preferred_element_type=jnp.float32)
        hb += pt_ref[...]
        hout_ref[...] = hb

        @pl.when(i == 0)
        def _():
            gh_ref[...] = jnp.full((NG, D), -jnp.inf, jnp.float32)

        bt = bt_ref[...]
        neg = jnp.full((BR, D), -jnp.inf, jnp.float32)
        parts = []
        for g in range(NG):
            vals = jnp.where(bt == g, hb, neg)
            parts.append(jnp.max(vals, axis=0, keepdims=True))
        gh_ref[...] = jnp.maximum(gh_ref[...], jnp.concatenate(parts, axis=0))

    return pl.pallas_call(
        body,
        grid=(N // BR,),
        in_specs=[
            pl.BlockSpec((BR, D), lambda i: (i, 0)),
            pl.BlockSpec((BR, D), lambda i: (i, 0)),
            pl.BlockSpec((D, D), lambda i: (0, 0)),
            pl.BlockSpec((BR, 1), lambda i: (i, 0)),
        ],
        out_specs=[
            pl.BlockSpec((BR, D), lambda i: (i, 0)),
            pl.BlockSpec((NG, D), lambda i: (0, 0)),
        ],
        out_shape=[
            jax.ShapeDtypeStruct((N, D), jnp.float32),
            jax.ShapeDtypeStruct((NG, D), jnp.float32),
        ],
    )(part, bw, w2t, batch.reshape(N, 1))


def kernel(h, fw_edge_index, bw_edge_index, batch, W_merge, b_merge):
    dstf, srcf = _prep_edges(fw_edge_index)
    dstb, srcb = _prep_edges(bw_edge_index)
    w1t = W_merge[:, :D].T
    w2t = W_merge[:, D:].T
    x = _sc_chain(h, dstf, srcf)
    part = _matmul1(x, w1t, b_merge)
    y = _sc_chain(h, dstb, srcb)
    h_out, g_h = _merge(part, y, w2t, batch)
    return (g_h, h_out)


# speedup vs baseline: 1.2000x; 1.2000x over previous
import functools

import jax
import jax.numpy as jnp
from jax import lax
from jax.experimental import pallas as pl
from jax.experimental.pallas import tpu as pltpu
from jax.experimental.pallas import tpu_sc as plsc

N = 10000
D = 256
HD = 128
E = 160000
CH = 128
NT = 16
NC = 2
NCH = E // CH
CPT0 = 79
NROW_PAD = N
RPT = 624
RPT_LAST = N - RPT * (NT - 1)
NG = 16
LAYERS = 3


def _prep_edges(ei):
    return jnp.ravel(ei[1]), jnp.ravel(ei[0])


def _sc_chain(x, dst3, src3):
    mesh = plsc.VectorSubcoreMesh(core_axis_name="c", subcore_axis_name="s")

    @functools.partial(
        pl.kernel,
        out_type=(jax.ShapeDtypeStruct((N, D), jnp.float32),
                  jax.ShapeDtypeStruct((N, D), jnp.float32),
                  jax.ShapeDtypeStruct((N, D), jnp.float32)),
        mesh=mesh,
        scratch_types=[
            pltpu.VMEM((CPT0 * CH,), jnp.int32),
            pltpu.VMEM((CH,), jnp.int32),
            pltpu.VMEM((CH,), jnp.int32),
            pltpu.VMEM_SHARED((NROW_PAD, HD), jnp.float32),
            pltpu.VMEM((CH, HD), jnp.float32),
            pltpu.VMEM((CH, HD), jnp.float32),
            pltpu.SemaphoreType.DMA,
            pltpu.SemaphoreType.DMA,
            pltpu.SemaphoreType.DMA,
            pltpu.SemaphoreType.DMA,
        ],
    )
    def chain(x_hbm, dst_hbm, src_hbm, out_hbm, t1_hbm, t2_hbm, idxg, srcb0,
              srcb1, acc, rows0, rows1, gsem0, gsem1, ssem0, ssem1):
        c = lax.axis_index("c")
        s = lax.axis_index("s")
        col = c * HD
        nc = 78 + (s < 2).astype(jnp.int32)
        ebase = (s * 78 + jnp.minimum(s, 2)) * CH

        @pl.when(s < 2)
        def _():
            pltpu.sync_copy(dst_hbm.at[pl.ds(ebase, 79 * CH)], idxg)

        @pl.when(s >= 2)
        def _():
            pltpu.sync_copy(dst_hbm.at[pl.ds(ebase, 78 * CH)],
                            idxg.at[pl.ds(0, 78 * CH)])

        def fetch_src(j, buf, sem):
            pltpu.async_copy(src_hbm.at[pl.ds(ebase + j * CH, CH)], buf, sem)

        def fetch_src_wait(j, buf, sem):
            pltpu.make_async_copy(
                src_hbm.at[pl.ds(ebase + j * CH, CH)], buf, sem).wait()

        @pl.when(s < NT - 1)
        def _():
            pltpu.sync_copy(x_hbm.at[pl.ds(s * RPT, RPT), pl.ds(col, HD)],
                            acc.at[pl.ds(s * RPT, RPT)])

        @pl.when(s == NT - 1)
        def _():
            pltpu.sync_copy(
                x_hbm.at[pl.ds((NT - 1) * RPT, RPT_LAST), pl.ds(col, HD)],
                acc.at[pl.ds((NT - 1) * RPT, RPT_LAST)])

        bufs = ((srcb0, rows0, gsem0, ssem0),
                (srcb1, rows1, gsem1, ssem1))

        def run_layer(xin_hbm, xout_hbm):
            def gather(j, buf, sem):
                return pltpu.async_copy(
                    xin_hbm.at[idxg.at[pl.ds(j * CH, CH)], pl.ds(col, HD)],
                    buf, sem)

            def gather_wait(j, buf, sem):
                pltpu.make_async_copy(
                    xin_hbm.at[idxg.at[pl.ds(j * CH, CH)], pl.ds(col, HD)],
                    buf, sem).wait()

            for k, (sb, rb, gs, ss) in enumerate(bufs):
                fetch_src(k, sb, ss)
                gather(k, rb, gs)
            plsc.subcore_barrier()

            @pl.loop(0, nc, step=2)
            def _(j):
                for k, (sb, rb, gs, ss) in enumerate(bufs):
                    @pl.when(j + k < nc)
                    def _(k=k, sb=sb, rb=rb, gs=gs, ss=ss):
                        fetch_src_wait(j + k, sb, ss)
                        gather_wait(j + k, rb, gs)
                        pltpu.sync_copy(rb, acc.at[sb], add=True)

                        @pl.when(j + k + 2 < nc)
                        def _():
                            fetch_src(j + k + 2, sb, ss)
                            gather(j + k + 2, rb, gs)

            plsc.subcore_barrier()

            @pl.when(s < NT - 1)
            def _():
                pltpu.sync_copy(
                    acc.at[pl.ds(s * RPT, RPT)],
                    xout_hbm.at[pl.ds(s * RPT, RPT), pl.ds(col, HD)])

            @pl.when(s == NT - 1)
            def _():
                pltpu.sync_copy(
                    acc.at[pl.ds((NT - 1) * RPT, RPT_LAST)],
                    xout_hbm.at[pl.ds((NT - 1) * RPT, RPT_LAST),
                                pl.ds(col, HD)])

            plsc.subcore_barrier()

        run_layer(x_hbm, t1_hbm)
        run_layer(t1_hbm, t2_hbm)
        run_layer(t2_hbm, out_hbm)

    return chain(x, dst3, src3)[0]


BR = 400


def _matmul1(fw, w1t, b):
    def body(fw_ref, w1_ref, b_ref, out_ref):
        out_ref[...] = jnp.dot(fw_ref[...], w1_ref[...],
                               preferred_element_type=jnp.float32) \
            + b_ref[...][None, :]

    return pl.pallas_call(
        body,
        grid=(N // BR,),
        in_specs=[
            pl.BlockSpec((BR, D), lambda i: (i, 0)),
            pl.BlockSpec((D, D), lambda i: (0, 0)),
            pl.BlockSpec((D,), lambda i: (0,)),
        ],
        out_specs=pl.BlockSpec((BR, D), lambda i: (i, 0)),
        out_shape=jax.ShapeDtypeStruct((N, D), jnp.float32),
    )(fw, w1t, b)


def _merge(part, bw, w2t, batch):
    def body(pt_ref, bw_ref, w2_ref, bt_ref, hout_ref, gh_ref):
        i = pl.program_id(0)
        hb = jnp.dot(bw_ref[...], w2_ref[...],
                     preferred_element_type=jnp.float32)
        hb += pt_ref[...]
        hout_ref[...] = hb

        @pl.when(i == 0)
        def _():
            gh_ref[...] = jnp.full((NG, D), -jnp.inf, jnp.float32)

        bt = bt_ref[...]
        neg = jnp.full((BR, D), -jnp.inf, jnp.float32)
        parts = []
        for g in range(NG):
            vals = jnp.where(bt == g, hb, neg)
            parts.append(jnp.max(vals, axis=0, keepdims=True))
        gh_ref[...] = jnp.maximum(gh_ref[...], jnp.concatenate(parts, axis=0))

    return pl.pallas_call(
        body,
        grid=(N // BR,),
        in_specs=[
            pl.BlockSpec((BR, D), lambda i: (i, 0)),
            pl.BlockSpec((BR, D), lambda i: (i, 0)),
            pl.BlockSpec((D, D), lambda i: (0, 0)),
            pl.BlockSpec((BR, 1), lambda i: (i, 0)),
        ],
        out_specs=[
            pl.BlockSpec((BR, D), lambda i: (i, 0)),
            pl.BlockSpec((NG, D), lambda i: (0, 0)),
        ],
        out_shape=[
            jax.ShapeDtypeStruct((N, D), jnp.float32),
            jax.ShapeDtypeStruct((NG, D), jnp.float32),
        ],
    )(part, bw, w2t, batch.reshape(N, 1))


def kernel(h, fw_edge_index, bw_edge_index, batch, W_merge, b_merge):
    dstf, srcf = _prep_edges(fw_edge_index)
    dstb, srcb = _prep_edges(bw_edge_index)
    w1t = W_merge[:, :D].T
    w2t = W_merge[:, D:].T
    x = _sc_chain(h, dstf, srcf)
    part = _matmul1(x, w1t, b_merge)
    y = _sc_chain(h, dstb, srcb)
    h_out, g_h = _merge(part, y, w2t, batch)
    return (g_h, h_out)


# speedup vs baseline: 1.2114x; 1.0095x over previous
import functools

import jax
import jax.numpy as jnp
from jax import lax
from jax.experimental import pallas as pl
from jax.experimental.pallas import tpu as pltpu
from jax.experimental.pallas import tpu_sc as plsc

N = 10000
D = 256
HD = 128
E = 160000
CH = 128
NT = 16
NC = 2
NCH = E // CH
CPT0 = 79
NROW_PAD = N
RPT = 624
RPT_LAST = N - RPT * (NT - 1)
NG = 16
LAYERS = 3


def _prep_edges(ei):
    return jnp.ravel(ei[1]), jnp.ravel(ei[0])


def _sc_chain(x, dst3, src3):
    mesh = plsc.VectorSubcoreMesh(core_axis_name="c", subcore_axis_name="s")

    @functools.partial(
        pl.kernel,
        out_type=(jax.ShapeDtypeStruct((N, D), jnp.float32),
                  jax.ShapeDtypeStruct((N, D), jnp.float32),
                  jax.ShapeDtypeStruct((N, D), jnp.float32)),
        mesh=mesh,
        scratch_types=[
            pltpu.VMEM((CPT0 * CH,), jnp.int32),
            pltpu.VMEM((CH,), jnp.int32),
            pltpu.VMEM((CH,), jnp.int32),
            pltpu.VMEM_SHARED((NROW_PAD, HD), jnp.float32),
            pltpu.VMEM((CH, HD), jnp.float32),
            pltpu.VMEM((CH, HD), jnp.float32),
            pltpu.SemaphoreType.DMA,
            pltpu.SemaphoreType.DMA,
            pltpu.SemaphoreType.DMA,
            pltpu.SemaphoreType.DMA,
        ],
    )
    def chain(x_hbm, dst_hbm, src_hbm, out_hbm, t1_hbm, t2_hbm, idxg, srcb0,
              srcb1, acc, rows0, rows1, gsem0, gsem1, ssem0, ssem1):
        c = lax.axis_index("c")
        s = lax.axis_index("s")
        col = c * HD
        nc = 78 + (s < 2).astype(jnp.int32)
        ebase = (s * 78 + jnp.minimum(s, 2)) * CH

        @pl.when(s < 2)
        def _():
            pltpu.sync_copy(dst_hbm.at[pl.ds(ebase, 79 * CH)], idxg)

        @pl.when(s >= 2)
        def _():
            pltpu.sync_copy(dst_hbm.at[pl.ds(ebase, 78 * CH)],
                            idxg.at[pl.ds(0, 78 * CH)])

        def fetch_src(j, buf, sem):
            pltpu.async_copy(src_hbm.at[pl.ds(ebase + j * CH, CH)], buf, sem)

        def fetch_src_wait(j, buf, sem):
            pltpu.make_async_copy(
                src_hbm.at[pl.ds(ebase + j * CH, CH)], buf, sem).wait()

        @pl.when(s < NT - 1)
        def _():
            pltpu.sync_copy(x_hbm.at[pl.ds(s * RPT, RPT), pl.ds(col, HD)],
                            acc.at[pl.ds(s * RPT, RPT)])

        @pl.when(s == NT - 1)
        def _():
            pltpu.sync_copy(
                x_hbm.at[pl.ds((NT - 1) * RPT, RPT_LAST), pl.ds(col, HD)],
                acc.at[pl.ds((NT - 1) * RPT, RPT_LAST)])

        bufs = ((srcb0, rows0, gsem0, ssem0),
                (srcb1, rows1, gsem1, ssem1))

        def run_layer(xin_hbm, xout_hbm):
            def gather(j, buf, sem):
                return pltpu.async_copy(
                    xin_hbm.at[idxg.at[pl.ds(j * CH, CH)], pl.ds(col, HD)],
                    buf, sem)

            def gather_wait(j, buf, sem):
                pltpu.make_async_copy(
                    xin_hbm.at[idxg.at[pl.ds(j * CH, CH)], pl.ds(col, HD)],
                    buf, sem).wait()

            for k, (sb, rb, gs, ss) in enumerate(bufs):
                fetch_src(k, sb, ss)
                gather(k, rb, gs)
            plsc.subcore_barrier()

            @pl.loop(0, nc, step=2)
            def _(j):
                for k, (sb, rb, gs, ss) in enumerate(bufs):
                    @pl.when(j + k < nc)
                    def _(k=k, sb=sb, rb=rb, gs=gs, ss=ss):
                        fetch_src_wait(j + k, sb, ss)
                        gather_wait(j + k, rb, gs)
                        pltpu.sync_copy(rb, acc.at[sb], add=True)

                        @pl.when(j + k + 2 < nc)
                        def _():
                            fetch_src(j + k + 2, sb, ss)
                            gather(j + k + 2, rb, gs)

            plsc.subcore_barrier()

            @pl.when(s < NT - 1)
            def _():
                pltpu.sync_copy(
                    acc.at[pl.ds(s * RPT, RPT)],
                    xout_hbm.at[pl.ds(s * RPT, RPT), pl.ds(col, HD)])

            @pl.when(s == NT - 1)
            def _():
                pltpu.sync_copy(
                    acc.at[pl.ds((NT - 1) * RPT, RPT_LAST)],
                    xout_hbm.at[pl.ds((NT - 1) * RPT, RPT_LAST),
                                pl.ds(col, HD)])

            plsc.subcore_barrier()

        run_layer(x_hbm, t1_hbm)
        run_layer(t1_hbm, t2_hbm)
        run_layer(t2_hbm, out_hbm)

    return chain(x, dst3, src3)[0]


BR = 400


def _matmul1(fw, w1t, b):
    def body(fw_ref, w1_ref, b_ref, out_ref):
        out_ref[...] = jnp.dot(fw_ref[...], w1_ref[...],
                               preferred_element_type=jnp.float32) \
            + b_ref[...][None, :]

    return pl.pallas_call(
        body,
        grid=(N // BR,),
        in_specs=[
            pl.BlockSpec((BR, D), lambda i: (i, 0)),
            pl.BlockSpec((D, D), lambda i: (0, 0)),
            pl.BlockSpec((D,), lambda i: (0,)),
        ],
        out_specs=pl.BlockSpec((BR, D), lambda i: (i, 0)),
        out_shape=jax.ShapeDtypeStruct((N, D), jnp.float32),
    )(fw, w1t, b)


def _merge(part, bw, w2t, batch):
    def body(pt_ref, bw_ref, w2_ref, bt_ref, hout_ref, gh_ref):
        i = pl.program_id(0)
        hb = jnp.dot(bw_ref[...], w2_ref[...],
                     preferred_element_type=jnp.float32)
        hb += pt_ref[...]
        hout_ref[...] = hb

        @pl.when(i == 0)
        def _():
            gh_ref[...] = jnp.full((NG, D), -jnp.inf, jnp.float32)

        bt = bt_ref[...]
        neg = jnp.full((BR, D), -jnp.inf, jnp.float32)

        def body_g(g, carry):
            vals = jnp.where(bt == g, hb, neg)
            m = jnp.max(vals, axis=0, keepdims=True)
            gh_ref[pl.ds(g, 1), :] = jnp.maximum(gh_ref[pl.ds(g, 1), :], m)
            return carry

        lax.fori_loop(bt_ref[0, 0], bt_ref[BR - 1, 0] + 1, body_g, 0)

    return pl.pallas_call(
        body,
        grid=(N // BR,),
        in_specs=[
            pl.BlockSpec((BR, D), lambda i: (i, 0)),
            pl.BlockSpec((BR, D), lambda i: (i, 0)),
            pl.BlockSpec((D, D), lambda i: (0, 0)),
            pl.BlockSpec((BR, 1), lambda i: (i, 0)),
        ],
        out_specs=[
            pl.BlockSpec((BR, D), lambda i: (i, 0)),
            pl.BlockSpec((NG, D), lambda i: (0, 0)),
        ],
        out_shape=[
            jax.ShapeDtypeStruct((N, D), jnp.float32),
            jax.ShapeDtypeStruct((NG, D), jnp.float32),
        ],
    )(part, bw, w2t, batch.reshape(N, 1))


def kernel(h, fw_edge_index, bw_edge_index, batch, W_merge, b_merge):
    dstf, srcf = _prep_edges(fw_edge_index)
    dstb, srcb = _prep_edges(bw_edge_index)
    w1t = W_merge[:, :D].T
    w2t = W_merge[:, D:].T
    x = _sc_chain(h, dstf, srcf)
    part = _matmul1(x, w1t, b_merge)
    y = _sc_chain(h, dstb, srcb)
    h_out, g_h = _merge(part, y, w2t, batch)
    return (g_h, h_out)
